# Initial kernel scaffold; baseline (speedup 1.0000x reference)
import jax
import jax.numpy as jnp
from jax.experimental import pallas as pl


def kernel(input_data, W_gate, W_experts, b_experts):
    raise NotImplementedError("write your pallas kernel here")


# speedup vs baseline: 1.9344x; 1.9344x over previous
import jax
import jax.numpy as jnp
from jax.experimental import pallas as pl

_N_EXPERTS = 8
_D_MODEL = 768
_N_TOKENS = 2048
_TB = 256


def _moe_block_kernel(x_ref, wg_ref, we_ref, be_ref, out_ref):
    x = x_ref[...]
    logits = jax.lax.dot_general(
        x, wg_ref[...], (((1,), (1,)), ((), ())),
        preferred_element_type=jnp.float32)
    g = jax.nn.softmax(logits, axis=1)
    e_iota = jax.lax.broadcasted_iota(jnp.int32, (_TB, _N_EXPERTS), 1)
    m1 = jnp.max(g, axis=1, keepdims=True)
    i1 = jnp.min(jnp.where(g == m1, e_iota, _N_EXPERTS), axis=1, keepdims=True)
    g2 = jnp.where(e_iota == i1, -jnp.inf, g)
    m2 = jnp.max(g2, axis=1, keepdims=True)
    i2 = jnp.min(jnp.where(g2 == m2, e_iota, _N_EXPERTS), axis=1, keepdims=True)
    gw = jnp.where((e_iota == i1) | (e_iota == i2), g, 0.0)

    xb = x.astype(jnp.bfloat16)
    be = be_ref[...]
    acc = jnp.zeros((_TB, _D_MODEL), jnp.float32)
    for e in range(_N_EXPERTS):
        ye = jax.lax.dot_general(
            xb, we_ref[e], (((1,), (1,)), ((), ())),
            preferred_element_type=jnp.float32)
        acc = acc + gw[:, e][:, None] * (ye + be[e][None, :])
    out_ref[...] = acc


def kernel(input_data, W_gate, W_experts, b_experts):
    we_bf16 = W_experts.astype(jnp.bfloat16)
    grid = (_N_TOKENS // _TB,)
    return pl.pallas_call(
        _moe_block_kernel,
        grid=grid,
        in_specs=[
            pl.BlockSpec((_TB, _D_MODEL), lambda i: (i, 0)),
            pl.BlockSpec((_N_EXPERTS, _D_MODEL), lambda i: (0, 0)),
            pl.BlockSpec((_N_EXPERTS, _D_MODEL, _D_MODEL), lambda i: (0, 0, 0)),
            pl.BlockSpec((_N_EXPERTS, _D_MODEL), lambda i: (0, 0)),
        ],
        out_specs=pl.BlockSpec((_TB, _D_MODEL), lambda i: (i, 0)),
        out_shape=jax.ShapeDtypeStruct((_N_TOKENS, _D_MODEL), jnp.float32),
    )(input_data, W_gate, we_bf16, b_experts)


# speedup vs baseline: 2.4202x; 1.2511x over previous
import jax
import jax.numpy as jnp
from jax.experimental import pallas as pl

_N_EXPERTS = 8
_D_MODEL = 768
_N_TOKENS = 2048
_TB = 1024


def _moe_block_kernel(x_ref, wg_ref, we_ref, be_ref, out_ref):
    x = x_ref[...]
    logits = jax.lax.dot_general(
        x, wg_ref[...], (((1,), (1,)), ((), ())),
        preferred_element_type=jnp.float32)
    g = jax.nn.softmax(logits, axis=1)
    e_iota = jax.lax.broadcasted_iota(jnp.int32, (_TB, _N_EXPERTS), 1)
    m1 = jnp.max(g, axis=1, keepdims=True)
    i1 = jnp.min(jnp.where(g == m1, e_iota, _N_EXPERTS), axis=1, keepdims=True)
    g2 = jnp.where(e_iota == i1, -jnp.inf, g)
    m2 = jnp.max(g2, axis=1, keepdims=True)
    i2 = jnp.min(jnp.where(g2 == m2, e_iota, _N_EXPERTS), axis=1, keepdims=True)
    gw = jnp.where((e_iota == i1) | (e_iota == i2), g, 0.0)

    xb = x.astype(jnp.bfloat16)
    be = be_ref[...]
    acc = jnp.zeros((_TB, _D_MODEL), jnp.float32)
    for e in range(_N_EXPERTS):
        ye = jax.lax.dot_general(
            xb, we_ref[e].astype(jnp.bfloat16), (((1,), (1,)), ((), ())),
            preferred_element_type=jnp.float32)
        acc = acc + gw[:, e][:, None] * (ye + be[e][None, :])
    out_ref[...] = acc


def kernel(input_data, W_gate, W_experts, b_experts):
    grid = (_N_TOKENS // _TB,)
    return pl.pallas_call(
        _moe_block_kernel,
        grid=grid,
        in_specs=[
            pl.BlockSpec((_TB, _D_MODEL), lambda i: (i, 0)),
            pl.BlockSpec((_N_EXPERTS, _D_MODEL), lambda i: (0, 0)),
            pl.BlockSpec((_N_EXPERTS, _D_MODEL, _D_MODEL), lambda i: (0, 0, 0)),
            pl.BlockSpec((_N_EXPERTS, _D_MODEL), lambda i: (0, 0)),
        ],
        out_specs=pl.BlockSpec((_TB, _D_MODEL), lambda i: (i, 0)),
        out_shape=jax.ShapeDtypeStruct((_N_TOKENS, _D_MODEL), jnp.float32),
    )(input_data, W_gate, W_experts, b_experts)
